# Initial kernel scaffold; baseline (speedup 1.0000x reference)
import jax
import jax.numpy as jnp
from jax.experimental import pallas as pl


def kernel(nodes_features_list, sp_mod_matrix_list, sp_adj_edge_index, W_gcn1, W_comm1, W_gcn2, W_comm2, W_out):
    raise NotImplementedError("write your pallas kernel here")


# speedup vs baseline: 3.3825x; 3.3825x over previous
import functools

import jax
import jax.numpy as jnp
from jax import lax
from jax.experimental import pallas as pl
from jax.experimental.pallas import tpu as pltpu
from jax.experimental.pallas import tpu_sc as plsc

N = 10000
E = 320000
D0 = 128
D1 = 128
D2 = 64

NC = 2
NS = 16
NW = NC * NS
CHUNK = 128
CH = -(-E // (NW * CHUNK))
E_PAD = NW * CH * CHUNK
ROWS_PER_TILE = 632
NP = NS * ROWS_PER_TILE


def _leaky(v):
    return jnp.where(v >= 0, v, 0.01 * v)


@functools.lru_cache(None)
def _make_agg():
    mesh = plsc.VectorSubcoreMesh(core_axis_name="c", subcore_axis_name="s")

    @functools.partial(
        pl.kernel,
        mesh=mesh,
        out_type=jax.ShapeDtypeStruct((NC, NP, D1), jnp.float32),
        scratch_types=[
            pltpu.VMEM((CH, CHUNK), jnp.int32),
            pltpu.VMEM((CH, CHUNK), jnp.int32),
            pltpu.VMEM((CHUNK, D1), jnp.float32),
            pltpu.VMEM_SHARED((NP, D1), jnp.float32),
            pltpu.SemaphoreType.DMA,
        ],
    )
    def agg(x_hbm, src_hbm, dst_hbm, zeros_hbm, out_hbm,
            src_v, dst_v, rows_v, acc_sh, sem):
        cid = lax.axis_index("c")
        sid = lax.axis_index("s")
        wid = cid * NS + sid
        row0 = pl.multiple_of(sid * ROWS_PER_TILE, 8)
        pltpu.sync_copy(zeros_hbm.at[pl.ds(row0, ROWS_PER_TILE)],
                        acc_sh.at[pl.ds(row0, ROWS_PER_TILE)])
        pltpu.sync_copy(src_hbm.at[wid], src_v)
        pltpu.sync_copy(dst_hbm.at[wid], dst_v)
        plsc.subcore_barrier()

        def chunk_body(j, carry):
            pltpu.async_copy(x_hbm.at[src_v.at[j]], rows_v, sem).wait()
            pltpu.sync_copy(rows_v, acc_sh.at[dst_v.at[j]], add=True)
            return carry

        lax.fori_loop(0, CH, chunk_body, 0)
        plsc.subcore_barrier()
        pltpu.sync_copy(acc_sh.at[pl.ds(row0, ROWS_PER_TILE)],
                        out_hbm.at[cid, pl.ds(row0, ROWS_PER_TILE)])

    return agg


RB1 = 400
G1 = N // RB1
RB = 400
G = N // RB


def _pc1_body(b_ref, wc_ref, x_ref, wg_ref, comm_ref, x1_ref):
    comm_ref[...] = _leaky(jnp.dot(b_ref[...], wc_ref[...],
                                   preferred_element_type=jnp.float32))
    x1_ref[...] = jnp.dot(x_ref[...], wg_ref[...],
                          preferred_element_type=jnp.float32)


def _pc1(B, Wc, X, Wg):
    return pl.pallas_call(
        _pc1_body,
        grid=(G1,),
        in_specs=[
            pl.BlockSpec((RB1, N), lambda i: (i, 0)),
            pl.BlockSpec((N, D1), lambda i: (0, 0)),
            pl.BlockSpec((RB1, D0), lambda i: (i, 0)),
            pl.BlockSpec((D0, D1), lambda i: (0, 0)),
        ],
        out_specs=[
            pl.BlockSpec((RB1, D1), lambda i: (i, 0)),
            pl.BlockSpec((RB1, D1), lambda i: (i, 0)),
        ],
        out_shape=[
            jax.ShapeDtypeStruct((N, D1), jnp.float32),
            jax.ShapeDtypeStruct((N, D1), jnp.float32),
        ],
        compiler_params=pltpu.CompilerParams(
            dimension_semantics=("parallel",)),
    )(B, Wc, X, Wg)


def _pc3_body(p_ref, comm1_ref, wg2_ref, wc2_ref, x2_ref, comm2_ref):
    h = _leaky(p_ref[0] + p_ref[1]) + comm1_ref[...]
    x2 = jnp.dot(h, wg2_ref[...], preferred_element_type=jnp.float32)
    x2_ref[...] = jnp.concatenate(
        [x2, jnp.zeros((RB, D1 - D2), jnp.float32)], axis=1)
    comm2_ref[...] = _leaky(jnp.dot(comm1_ref[...], wc2_ref[...],
                                    preferred_element_type=jnp.float32))


def _pc3(p, comm1, Wg2, Wc2):
    return pl.pallas_call(
        _pc3_body,
        grid=(G,),
        in_specs=[
            pl.BlockSpec((NC, RB, D1), lambda i: (0, i, 0)),
            pl.BlockSpec((RB, D1), lambda i: (i, 0)),
            pl.BlockSpec((D1, D2), lambda i: (0, 0)),
            pl.BlockSpec((D1, D2), lambda i: (0, 0)),
        ],
        out_specs=[
            pl.BlockSpec((RB, D1), lambda i: (i, 0)),
            pl.BlockSpec((RB, D2), lambda i: (i, 0)),
        ],
        out_shape=[
            jax.ShapeDtypeStruct((NP, D1), jnp.float32),
            jax.ShapeDtypeStruct((N, D2), jnp.float32),
        ],
        compiler_params=pltpu.CompilerParams(
            dimension_semantics=("parallel",)),
    )(p, comm1, Wg2, Wc2)


def _pc5_body(q_ref, comm2_ref, wout_ref, x3_ref):
    h = _leaky(q_ref[0, :, :D2] + q_ref[1, :, :D2]) + comm2_ref[...]
    x3 = jnp.dot(h, wout_ref[...], preferred_element_type=jnp.float32)
    x3_ref[...] = jnp.concatenate(
        [x3, jnp.zeros((RB, D1 - D2), jnp.float32)], axis=1)


def _pc5(q, comm2, Wout):
    return pl.pallas_call(
        _pc5_body,
        grid=(G,),
        in_specs=[
            pl.BlockSpec((NC, RB, D1), lambda i: (0, i, 0)),
            pl.BlockSpec((RB, D2), lambda i: (i, 0)),
            pl.BlockSpec((D2, D2), lambda i: (0, 0)),
        ],
        out_specs=pl.BlockSpec((RB, D1), lambda i: (i, 0)),
        out_shape=jax.ShapeDtypeStruct((NP, D1), jnp.float32),
        compiler_params=pltpu.CompilerParams(
            dimension_semantics=("parallel",)),
    )(q, comm2, Wout)


def _pc7_body(r_ref, out_ref):
    out_ref[...] = _leaky(r_ref[0, :, :D2] + r_ref[1, :, :D2])


def _pc7(r):
    return pl.pallas_call(
        _pc7_body,
        grid=(G,),
        in_specs=[pl.BlockSpec((NC, RB, D1), lambda i: (0, i, 0))],
        out_specs=pl.BlockSpec((RB, D2), lambda i: (i, 0)),
        out_shape=jax.ShapeDtypeStruct((N, D2), jnp.float32),
        compiler_params=pltpu.CompilerParams(
            dimension_semantics=("parallel",)),
    )(r)


def kernel(nodes_features_list, sp_mod_matrix_list, sp_adj_edge_index,
           W_gcn1, W_comm1, W_gcn2, W_comm2, W_out):
    src = sp_adj_edge_index[0]
    dst = sp_adj_edge_index[1]
    pad = E_PAD - E
    src_p = jnp.concatenate([src, jnp.zeros((pad,), jnp.int32)]
                            ).reshape(NW, CH, CHUNK)
    dst_p = jnp.concatenate([dst, jnp.full((pad,), N, jnp.int32)]
                            ).reshape(NW, CH, CHUNK)
    zeros128 = jnp.zeros((NP, D1), jnp.float32)

    agg = _make_agg()
    comm1, x1 = _pc1(sp_mod_matrix_list, W_comm1, nodes_features_list, W_gcn1)
    p1 = agg(x1, src_p, dst_p, zeros128)
    x2, comm2 = _pc3(p1, comm1, W_gcn2, W_comm2)
    p2 = agg(x2, src_p, dst_p, zeros128)
    x3 = _pc5(p2, comm2, W_out)
    p3 = agg(x3, src_p, dst_p, zeros128)
    h = _pc7(p3)
    return (h, comm2)


# speedup vs baseline: 7.7756x; 2.2988x over previous
import functools

import jax
import jax.numpy as jnp
from jax import lax
from jax.experimental import pallas as pl
from jax.experimental.pallas import tpu as pltpu
from jax.experimental.pallas import tpu_sc as plsc

N = 10000
E = 320000
D0 = 128
D1 = 128
D2 = 64

NC = 2
NS = 16
NW = NC * NS
CHUNK = 128
RIDX = 4
CH = 80
E_PAD = NW * CH * CHUNK
ROWS_PER_TILE = 632
NP = NS * ROWS_PER_TILE


def _leaky(v):
    return jnp.where(v >= 0, v, 0.01 * v)


@functools.lru_cache(None)
def _make_agg():
    mesh = plsc.VectorSubcoreMesh(core_axis_name="c", subcore_axis_name="s")

    @functools.partial(
        pl.kernel,
        mesh=mesh,
        out_type=jax.ShapeDtypeStruct((NC, NP, D1), jnp.float32),
        scratch_types=[
            pltpu.VMEM((RIDX, 2, CHUNK), jnp.int32),
            pltpu.VMEM((2, CHUNK, D1), jnp.float32),
            pltpu.VMEM_SHARED((NP, D1), jnp.float32),
        ] + [pltpu.SemaphoreType.DMA] * (2 + RIDX),
    )
    def agg(x_hbm, e_hbm, zeros_hbm, out_hbm,
            idx_v, rows_v, acc_sh, *sems):
        gsem = sems[:2]
        isem = sems[2:]
        cid = lax.axis_index("c")
        sid = lax.axis_index("s")
        wid = cid * NS + sid
        row0 = pl.multiple_of(sid * ROWS_PER_TILE, 8)
        pltpu.sync_copy(zeros_hbm.at[pl.ds(row0, ROWS_PER_TILE)],
                        acc_sh.at[pl.ds(row0, ROWS_PER_TILE)])
        plsc.subcore_barrier()

        for r in range(RIDX):
            pltpu.async_copy(e_hbm.at[wid, r], idx_v.at[r], isem[r])
        for b in range(2):
            pltpu.make_async_copy(
                e_hbm.at[wid, b], idx_v.at[b], isem[b]).wait()
            pltpu.async_copy(
                x_hbm.at[idx_v.at[b, 0]], rows_v.at[b], gsem[b])

        def group(g, carry):
            j0 = g * RIDX
            for u in range(RIDX):
                j = j0 + u
                b = u % 2
                r = u
                pltpu.make_async_copy(
                    x_hbm.at[idx_v.at[r, 0]], rows_v.at[b],
                    gsem[b]).wait()
                pltpu.sync_copy(rows_v.at[b], acc_sh.at[idx_v.at[r, 1]],
                                add=True)
                jn2 = j + 2
                rn2 = (u + 2) % RIDX

                @pl.when(jn2 < CH)
                def _start_gather(b=b, rn2=rn2, jn2=jn2):
                    pltpu.make_async_copy(
                        e_hbm.at[wid, jn2], idx_v.at[rn2],
                        isem[rn2]).wait()
                    pltpu.async_copy(
                        x_hbm.at[idx_v.at[rn2, 0]], rows_v.at[b], gsem[b])

                jn4 = j + RIDX

                @pl.when(jn4 < CH)
                def _start_idx(r=r, jn4=jn4):
                    pltpu.async_copy(
                        e_hbm.at[wid, jn4], idx_v.at[r], isem[r])
            return carry

        lax.fori_loop(0, CH // RIDX, group, 0)
        plsc.subcore_barrier()
        pltpu.sync_copy(acc_sh.at[pl.ds(row0, ROWS_PER_TILE)],
                        out_hbm.at[cid, pl.ds(row0, ROWS_PER_TILE)])

    return agg


RB1 = 400
G1 = N // RB1
RB = 400
G = N // RB


def _pc1_body(b_ref, wc_ref, x_ref, wg_ref, comm_ref, x1_ref):
    comm_ref[...] = _leaky(jnp.dot(b_ref[...], wc_ref[...],
                                   preferred_element_type=jnp.float32))
    x1_ref[...] = jnp.dot(x_ref[...], wg_ref[...],
                          preferred_element_type=jnp.float32)


def _pc1(B, Wc, X, Wg):
    return pl.pallas_call(
        _pc1_body,
        grid=(G1,),
        in_specs=[
            pl.BlockSpec((RB1, N), lambda i: (i, 0)),
            pl.BlockSpec((N, D1), lambda i: (0, 0)),
            pl.BlockSpec((RB1, D0), lambda i: (i, 0)),
            pl.BlockSpec((D0, D1), lambda i: (0, 0)),
        ],
        out_specs=[
            pl.BlockSpec((RB1, D1), lambda i: (i, 0)),
            pl.BlockSpec((RB1, D1), lambda i: (i, 0)),
        ],
        out_shape=[
            jax.ShapeDtypeStruct((N, D1), jnp.float32),
            jax.ShapeDtypeStruct((NP, D1), jnp.float32),
        ],
        compiler_params=pltpu.CompilerParams(
            dimension_semantics=("parallel",)),
    )(B, Wc, X, Wg)


def _pc3_body(p_ref, comm1_ref, wg2_ref, wc2_ref, x2_ref, comm2_ref):
    h = _leaky(p_ref[0] + p_ref[1]) + comm1_ref[...]
    x2 = jnp.dot(h, wg2_ref[...], preferred_element_type=jnp.float32)
    x2_ref[...] = jnp.concatenate(
        [x2, jnp.zeros((RB, D1 - D2), jnp.float32)], axis=1)
    comm2_ref[...] = _leaky(jnp.dot(comm1_ref[...], wc2_ref[...],
                                    preferred_element_type=jnp.float32))


def _pc3(p, comm1, Wg2, Wc2):
    return pl.pallas_call(
        _pc3_body,
        grid=(G,),
        in_specs=[
            pl.BlockSpec((NC, RB, D1), lambda i: (0, i, 0)),
            pl.BlockSpec((RB, D1), lambda i: (i, 0)),
            pl.BlockSpec((D1, D2), lambda i: (0, 0)),
            pl.BlockSpec((D1, D2), lambda i: (0, 0)),
        ],
        out_specs=[
            pl.BlockSpec((RB, D1), lambda i: (i, 0)),
            pl.BlockSpec((RB, D2), lambda i: (i, 0)),
        ],
        out_shape=[
            jax.ShapeDtypeStruct((NP, D1), jnp.float32),
            jax.ShapeDtypeStruct((N, D2), jnp.float32),
        ],
        compiler_params=pltpu.CompilerParams(
            dimension_semantics=("parallel",)),
    )(p, comm1, Wg2, Wc2)


def _pc5_body(q_ref, comm2_ref, wout_ref, x3_ref):
    h = _leaky(q_ref[0, :, :D2] + q_ref[1, :, :D2]) + comm2_ref[...]
    x3 = jnp.dot(h, wout_ref[...], preferred_element_type=jnp.float32)
    x3_ref[...] = jnp.concatenate(
        [x3, jnp.zeros((RB, D1 - D2), jnp.float32)], axis=1)


def _pc5(q, comm2, Wout):
    return pl.pallas_call(
        _pc5_body,
        grid=(G,),
        in_specs=[
            pl.BlockSpec((NC, RB, D1), lambda i: (0, i, 0)),
            pl.BlockSpec((RB, D2), lambda i: (i, 0)),
            pl.BlockSpec((D2, D2), lambda i: (0, 0)),
        ],
        out_specs=pl.BlockSpec((RB, D1), lambda i: (i, 0)),
        out_shape=jax.ShapeDtypeStruct((NP, D1), jnp.float32),
        compiler_params=pltpu.CompilerParams(
            dimension_semantics=("parallel",)),
    )(q, comm2, Wout)


def _pc7_body(r_ref, out_ref):
    out_ref[...] = _leaky(r_ref[0, :, :D2] + r_ref[1, :, :D2])


def _pc7(r):
    return pl.pallas_call(
        _pc7_body,
        grid=(G,),
        in_specs=[pl.BlockSpec((NC, RB, D1), lambda i: (0, i, 0))],
        out_specs=pl.BlockSpec((RB, D2), lambda i: (i, 0)),
        out_shape=jax.ShapeDtypeStruct((N, D2), jnp.float32),
        compiler_params=pltpu.CompilerParams(
            dimension_semantics=("parallel",)),
    )(r)


def kernel(nodes_features_list, sp_mod_matrix_list, sp_adj_edge_index,
           W_gcn1, W_comm1, W_gcn2, W_comm2, W_out):
    src = sp_adj_edge_index[0]
    dst = sp_adj_edge_index[1]
    pad = E_PAD - E
    it = jnp.arange(pad, dtype=jnp.int32)
    pad_src = (it * 131) % N
    pad_dst = N + it % (NP - N)
    src_p = jnp.concatenate([src, pad_src]).reshape(NW, CH, 1, CHUNK)
    dst_p = jnp.concatenate([dst, pad_dst]).reshape(NW, CH, 1, CHUNK)
    edges_p = jnp.concatenate([src_p, dst_p], axis=2)
    zeros128 = jnp.zeros((NP, D1), jnp.float32)

    agg = _make_agg()
    comm1, x1 = _pc1(sp_mod_matrix_list, W_comm1, nodes_features_list, W_gcn1)
    p1 = agg(x1, edges_p, zeros128)
    x2, comm2 = _pc3(p1, comm1, W_gcn2, W_comm2)
    p2 = agg(x2, edges_p, zeros128)
    x3 = _pc5(p2, comm2, W_out)
    p3 = agg(x3, edges_p, zeros128)
    h = _pc7(p3)
    return (h, comm2)


# speedup vs baseline: 8.2474x; 1.0607x over previous
import functools

import jax
import jax.numpy as jnp
from jax import lax
from jax.experimental import pallas as pl
from jax.experimental.pallas import tpu as pltpu
from jax.experimental.pallas import tpu_sc as plsc

N = 10000
E = 320000
D0 = 128
D1 = 128
D2 = 64

NC = 2
NS = 16
NW = NC * NS
CHUNK = 128
RIDX = 4
CH = 80
E_PAD = NW * CH * CHUNK
ROWS_PER_TILE = 632
NP = NS * ROWS_PER_TILE


def _leaky(v):
    return jnp.where(v >= 0, v, 0.01 * v)


@functools.lru_cache(None)
def _make_agg():
    mesh = plsc.VectorSubcoreMesh(core_axis_name="c", subcore_axis_name="s")

    @functools.partial(
        pl.kernel,
        mesh=mesh,
        out_type=jax.ShapeDtypeStruct((NC, NP, D1), jnp.float32),
        scratch_types=[
            pltpu.VMEM((RIDX, 2, CHUNK), jnp.int32),
            pltpu.VMEM((2, CHUNK, D1), jnp.float32),
            pltpu.VMEM_SHARED((NP, D1), jnp.float32),
        ] + [pltpu.SemaphoreType.DMA] * (2 + RIDX),
    )
    def agg(x_hbm, e_hbm, zeros_hbm, out_hbm,
            idx_v, rows_v, acc_sh, *sems):
        gsem = sems[:2]
        isem = sems[2:]
        cid = lax.axis_index("c")
        sid = lax.axis_index("s")
        wid = cid * NS + sid
        row0 = pl.multiple_of(sid * ROWS_PER_TILE, 8)
        pltpu.sync_copy(zeros_hbm.at[pl.ds(row0, ROWS_PER_TILE)],
                        acc_sh.at[pl.ds(row0, ROWS_PER_TILE)])
        plsc.subcore_barrier()

        for r in range(RIDX):
            pltpu.async_copy(e_hbm.at[wid, r], idx_v.at[r], isem[r])
        for b in range(2):
            pltpu.make_async_copy(
                e_hbm.at[wid, b], idx_v.at[b], isem[b]).wait()
            pltpu.async_copy(
                x_hbm.at[idx_v.at[b, 0]], rows_v.at[b], gsem[b])

        def group(g, carry):
            j0 = g * RIDX
            for u in range(RIDX):
                j = j0 + u
                b = u % 2
                r = u
                pltpu.make_async_copy(
                    x_hbm.at[idx_v.at[r, 0]], rows_v.at[b],
                    gsem[b]).wait()
                pltpu.sync_copy(rows_v.at[b], acc_sh.at[idx_v.at[r, 1]],
                                add=True)
                jn2 = j + 2
                rn2 = (u + 2) % RIDX

                @pl.when(jn2 < CH)
                def _start_gather(b=b, rn2=rn2, jn2=jn2):
                    pltpu.make_async_copy(
                        e_hbm.at[wid, jn2], idx_v.at[rn2],
                        isem[rn2]).wait()
                    pltpu.async_copy(
                        x_hbm.at[idx_v.at[rn2, 0]], rows_v.at[b], gsem[b])

                jn4 = j + RIDX

                @pl.when(jn4 < CH)
                def _start_idx(r=r, jn4=jn4):
                    pltpu.async_copy(
                        e_hbm.at[wid, jn4], idx_v.at[r], isem[r])
            return carry

        lax.fori_loop(0, CH // RIDX, group, 0)
        plsc.subcore_barrier()
        pltpu.sync_copy(acc_sh.at[pl.ds(row0, ROWS_PER_TILE)],
                        out_hbm.at[cid, pl.ds(row0, ROWS_PER_TILE)])

    return agg


RB1 = 400
G1 = N // RB1
RB = 400
G = N // RB


def _pc0_body(x_ref, wg_ref, x1_ref):
    x1_ref[...] = jnp.dot(x_ref[...], wg_ref[...],
                          preferred_element_type=jnp.float32)


def _pc0(X, Wg):
    return pl.pallas_call(
        _pc0_body,
        grid=(G1,),
        in_specs=[
            pl.BlockSpec((RB1, D0), lambda i: (i, 0)),
            pl.BlockSpec((D0, D1), lambda i: (0, 0)),
        ],
        out_specs=pl.BlockSpec((RB1, D1), lambda i: (i, 0)),
        out_shape=jax.ShapeDtypeStruct((NP, D1), jnp.float32),
        compiler_params=pltpu.CompilerParams(
            dimension_semantics=("parallel",)),
    )(X, Wg)


def _pc1_body(b_ref, wc_ref, comm_ref):
    comm_ref[...] = _leaky(jnp.dot(b_ref[...], wc_ref[...],
                                   preferred_element_type=jnp.float32))


def _pc1(B, Wc):
    return pl.pallas_call(
        _pc1_body,
        grid=(G1,),
        in_specs=[
            pl.BlockSpec((RB1, N), lambda i: (i, 0)),
            pl.BlockSpec((N, D1), lambda i: (0, 0)),
        ],
        out_specs=pl.BlockSpec((RB1, D1), lambda i: (i, 0)),
        out_shape=jax.ShapeDtypeStruct((N, D1), jnp.float32),
        compiler_params=pltpu.CompilerParams(
            dimension_semantics=("parallel",)),
    )(B, Wc)


def _pc3_body(p_ref, comm1_ref, wg2_ref, wc2_ref, x2_ref, comm2_ref):
    h = _leaky(p_ref[0] + p_ref[1]) + comm1_ref[...]
    x2 = jnp.dot(h, wg2_ref[...], preferred_element_type=jnp.float32)
    x2_ref[...] = jnp.concatenate(
        [x2, jnp.zeros((RB, D1 - D2), jnp.float32)], axis=1)
    comm2_ref[...] = _leaky(jnp.dot(comm1_ref[...], wc2_ref[...],
                                    preferred_element_type=jnp.float32))


def _pc3(p, comm1, Wg2, Wc2):
    return pl.pallas_call(
        _pc3_body,
        grid=(G,),
        in_specs=[
            pl.BlockSpec((NC, RB, D1), lambda i: (0, i, 0)),
            pl.BlockSpec((RB, D1), lambda i: (i, 0)),
            pl.BlockSpec((D1, D2), lambda i: (0, 0)),
            pl.BlockSpec((D1, D2), lambda i: (0, 0)),
        ],
        out_specs=[
            pl.BlockSpec((RB, D1), lambda i: (i, 0)),
            pl.BlockSpec((RB, D2), lambda i: (i, 0)),
        ],
        out_shape=[
            jax.ShapeDtypeStruct((NP, D1), jnp.float32),
            jax.ShapeDtypeStruct((N, D2), jnp.float32),
        ],
        compiler_params=pltpu.CompilerParams(
            dimension_semantics=("parallel",)),
    )(p, comm1, Wg2, Wc2)


def _pc5_body(q_ref, comm2_ref, wout_ref, x3_ref):
    h = _leaky(q_ref[0, :, :D2] + q_ref[1, :, :D2]) + comm2_ref[...]
    x3 = jnp.dot(h, wout_ref[...], preferred_element_type=jnp.float32)
    x3_ref[...] = jnp.concatenate(
        [x3, jnp.zeros((RB, D1 - D2), jnp.float32)], axis=1)


def _pc5(q, comm2, Wout):
    return pl.pallas_call(
        _pc5_body,
        grid=(G,),
        in_specs=[
            pl.BlockSpec((NC, RB, D1), lambda i: (0, i, 0)),
            pl.BlockSpec((RB, D2), lambda i: (i, 0)),
            pl.BlockSpec((D2, D2), lambda i: (0, 0)),
        ],
        out_specs=pl.BlockSpec((RB, D1), lambda i: (i, 0)),
        out_shape=jax.ShapeDtypeStruct((NP, D1), jnp.float32),
        compiler_params=pltpu.CompilerParams(
            dimension_semantics=("parallel",)),
    )(q, comm2, Wout)


def _pc7_body(r_ref, out_ref):
    out_ref[...] = _leaky(r_ref[0, :, :D2] + r_ref[1, :, :D2])


def _pc7(r):
    return pl.pallas_call(
        _pc7_body,
        grid=(G,),
        in_specs=[pl.BlockSpec((NC, RB, D1), lambda i: (0, i, 0))],
        out_specs=pl.BlockSpec((RB, D2), lambda i: (i, 0)),
        out_shape=jax.ShapeDtypeStruct((N, D2), jnp.float32),
        compiler_params=pltpu.CompilerParams(
            dimension_semantics=("parallel",)),
    )(r)


def kernel(nodes_features_list, sp_mod_matrix_list, sp_adj_edge_index,
           W_gcn1, W_comm1, W_gcn2, W_comm2, W_out):
    src = sp_adj_edge_index[0]
    dst = sp_adj_edge_index[1]
    pad = E_PAD - E
    it = jnp.arange(pad, dtype=jnp.int32)
    pad_src = (it * 131) % N
    pad_dst = N + it % (NP - N)
    src_p = jnp.concatenate([src, pad_src]).reshape(NW, CH, 1, CHUNK)
    dst_p = jnp.concatenate([dst, pad_dst]).reshape(NW, CH, 1, CHUNK)
    edges_p = jnp.concatenate([src_p, dst_p], axis=2)
    zeros128 = jnp.zeros((NP, D1), jnp.float32)

    agg = _make_agg()
    x1 = _pc0(nodes_features_list, W_gcn1)
    p1 = agg(x1, edges_p, zeros128)
    comm1 = _pc1(sp_mod_matrix_list, W_comm1)
    x2, comm2 = _pc3(p1, comm1, W_gcn2, W_comm2)
    p2 = agg(x2, edges_p, zeros128)
    x3 = _pc5(p2, comm2, W_out)
    p3 = agg(x3, edges_p, zeros128)
    h = _pc7(p3)
    return (h, comm2)


# speedup vs baseline: 8.4068x; 1.0193x over previous
import functools

import jax
import jax.numpy as jnp
from jax import lax
from jax.experimental import pallas as pl
from jax.experimental.pallas import tpu as pltpu
from jax.experimental.pallas import tpu_sc as plsc

N = 10000
E = 320000
D0 = 128
D1 = 128
D2 = 64

NC = 2
NS = 16
NW = NC * NS
CHUNK = 96
RROW = 3
RIDX = 6
CH = 108
E_PAD = NW * CH * CHUNK
ROWS_PER_TILE = 632
NP = NS * ROWS_PER_TILE


def _leaky(v):
    return jnp.where(v >= 0, v, 0.01 * v)


@functools.lru_cache(None)
def _make_agg():
    mesh = plsc.VectorSubcoreMesh(core_axis_name="c", subcore_axis_name="s")

    @functools.partial(
        pl.kernel,
        mesh=mesh,
        out_type=jax.ShapeDtypeStruct((NC, NP, D1), jnp.float32),
        scratch_types=[
            pltpu.VMEM((RIDX, 2, CHUNK), jnp.int32),
            pltpu.VMEM((RROW, CHUNK, D1), jnp.float32),
            pltpu.VMEM_SHARED((NP, D1), jnp.float32),
        ] + [pltpu.SemaphoreType.DMA] * (2 * RROW + RIDX),
    )
    def agg(x_hbm, e_hbm, zeros_hbm, out_hbm,
            idx_v, rows_v, acc_sh, *sems):
        gsem = sems[:RROW]
        ssem = sems[RROW:2 * RROW]
        isem = sems[2 * RROW:]
        cid = lax.axis_index("c")
        sid = lax.axis_index("s")
        wid = cid * NS + sid
        row0 = pl.multiple_of(sid * ROWS_PER_TILE, 8)
        pltpu.sync_copy(zeros_hbm.at[pl.ds(row0, ROWS_PER_TILE)],
                        acc_sh.at[pl.ds(row0, ROWS_PER_TILE)])
        plsc.subcore_barrier()

        def wait_i(j, r):
            pltpu.make_async_copy(e_hbm.at[wid, j], idx_v.at[r],
                                  isem[r]).wait()

        def start_i(j, r):
            pltpu.async_copy(e_hbm.at[wid, j], idx_v.at[r], isem[r])

        def start_g(r, b):
            pltpu.async_copy(x_hbm.at[idx_v.at[r, 0]], rows_v.at[b],
                             gsem[b])

        def wait_g(r, b):
            pltpu.make_async_copy(x_hbm.at[idx_v.at[r, 0]], rows_v.at[b],
                                  gsem[b]).wait()

        def start_s(r, b):
            pltpu.async_copy(rows_v.at[b], acc_sh.at[idx_v.at[r, 1]],
                             ssem[b], add=True)

        def wait_s(r, b):
            pltpu.make_async_copy(rows_v.at[b], acc_sh.at[idx_v.at[r, 1]],
                                  ssem[b]).wait()

        for j in range(RROW):
            start_i(j, j)

        def group(g, carry):
            j0 = g * RIDX
            for u in range(RIDX):
                j = j0 + u
                ri = u
                b = u % RROW
                wait_i(j, ri)

                @pl.when(j >= RROW)
                def _ws(ri=ri, b=b):
                    pltpu.make_async_copy(
                        rows_v.at[b], acc_sh.at[idx_v.at[ri, 1]],
                        ssem[b]).wait()

                jn = j + RROW
                rin = (u + RROW) % RIDX

                @pl.when(jn < CH)
                def _si(jn=jn, rin=rin):
                    start_i(jn, rin)

                start_g(ri, b)

                @pl.when(j >= 2)
                def _ds(u=u, b2=(u - 2) % RROW, ri2=(u - 2) % RIDX):
                    wait_g(ri2, b2)
                    start_s(ri2, b2)
            return carry

        lax.fori_loop(0, CH // RIDX, group, 0)
        for j in (CH - 2, CH - 1):
            ri = j % RIDX
            b = j % RROW
            wait_g(ri, b)
            start_s(ri, b)
        for j in (CH - 3, CH - 2, CH - 1):
            wait_s(j % RIDX, j % RROW)
        plsc.subcore_barrier()
        pltpu.sync_copy(acc_sh.at[pl.ds(row0, ROWS_PER_TILE)],
                        out_hbm.at[cid, pl.ds(row0, ROWS_PER_TILE)])

    return agg


RB1 = 400
G1 = N // RB1
RB = 400
G = N // RB


def _pc0_body(x_ref, wg_ref, x1_ref):
    x1_ref[...] = jnp.dot(x_ref[...], wg_ref[...],
                          preferred_element_type=jnp.float32)


def _pc0(X, Wg):
    return pl.pallas_call(
        _pc0_body,
        grid=(G1,),
        in_specs=[
            pl.BlockSpec((RB1, D0), lambda i: (i, 0)),
            pl.BlockSpec((D0, D1), lambda i: (0, 0)),
        ],
        out_specs=pl.BlockSpec((RB1, D1), lambda i: (i, 0)),
        out_shape=jax.ShapeDtypeStruct((NP, D1), jnp.float32),
        compiler_params=pltpu.CompilerParams(
            dimension_semantics=("parallel",)),
    )(X, Wg)


def _pc1_body(b_ref, wc_ref, comm_ref):
    comm_ref[...] = _leaky(jnp.dot(b_ref[...], wc_ref[...],
                                   preferred_element_type=jnp.float32))


def _pc1(B, Wc):
    return pl.pallas_call(
        _pc1_body,
        grid=(G1,),
        in_specs=[
            pl.BlockSpec((RB1, N), lambda i: (i, 0)),
            pl.BlockSpec((N, D1), lambda i: (0, 0)),
        ],
        out_specs=pl.BlockSpec((RB1, D1), lambda i: (i, 0)),
        out_shape=jax.ShapeDtypeStruct((N, D1), jnp.float32),
        compiler_params=pltpu.CompilerParams(
            dimension_semantics=("parallel",)),
    )(B, Wc)


def _pc3_body(p_ref, comm1_ref, wg2_ref, wc2_ref, x2_ref, comm2_ref):
    h = _leaky(p_ref[0] + p_ref[1]) + comm1_ref[...]
    x2 = jnp.dot(h, wg2_ref[...], preferred_element_type=jnp.float32)
    x2_ref[...] = jnp.concatenate(
        [x2, jnp.zeros((RB, D1 - D2), jnp.float32)], axis=1)
    comm2_ref[...] = _leaky(jnp.dot(comm1_ref[...], wc2_ref[...],
                                    preferred_element_type=jnp.float32))


def _pc3(p, comm1, Wg2, Wc2):
    return pl.pallas_call(
        _pc3_body,
        grid=(G,),
        in_specs=[
            pl.BlockSpec((NC, RB, D1), lambda i: (0, i, 0)),
            pl.BlockSpec((RB, D1), lambda i: (i, 0)),
            pl.BlockSpec((D1, D2), lambda i: (0, 0)),
            pl.BlockSpec((D1, D2), lambda i: (0, 0)),
        ],
        out_specs=[
            pl.BlockSpec((RB, D1), lambda i: (i, 0)),
            pl.BlockSpec((RB, D2), lambda i: (i, 0)),
        ],
        out_shape=[
            jax.ShapeDtypeStruct((NP, D1), jnp.float32),
            jax.ShapeDtypeStruct((N, D2), jnp.float32),
        ],
        compiler_params=pltpu.CompilerParams(
            dimension_semantics=("parallel",)),
    )(p, comm1, Wg2, Wc2)


def _pc5_body(q_ref, comm2_ref, wout_ref, x3_ref):
    h = _leaky(q_ref[0, :, :D2] + q_ref[1, :, :D2]) + comm2_ref[...]
    x3 = jnp.dot(h, wout_ref[...], preferred_element_type=jnp.float32)
    x3_ref[...] = jnp.concatenate(
        [x3, jnp.zeros((RB, D1 - D2), jnp.float32)], axis=1)


def _pc5(q, comm2, Wout):
    return pl.pallas_call(
        _pc5_body,
        grid=(G,),
        in_specs=[
            pl.BlockSpec((NC, RB, D1), lambda i: (0, i, 0)),
            pl.BlockSpec((RB, D2), lambda i: (i, 0)),
            pl.BlockSpec((D2, D2), lambda i: (0, 0)),
        ],
        out_specs=pl.BlockSpec((RB, D1), lambda i: (i, 0)),
        out_shape=jax.ShapeDtypeStruct((NP, D1), jnp.float32),
        compiler_params=pltpu.CompilerParams(
            dimension_semantics=("parallel",)),
    )(q, comm2, Wout)


def _pc7_body(r_ref, out_ref):
    out_ref[...] = _leaky(r_ref[0, :, :D2] + r_ref[1, :, :D2])


def _pc7(r):
    return pl.pallas_call(
        _pc7_body,
        grid=(G,),
        in_specs=[pl.BlockSpec((NC, RB, D1), lambda i: (0, i, 0))],
        out_specs=pl.BlockSpec((RB, D2), lambda i: (i, 0)),
        out_shape=jax.ShapeDtypeStruct((N, D2), jnp.float32),
        compiler_params=pltpu.CompilerParams(
            dimension_semantics=("parallel",)),
    )(r)


def kernel(nodes_features_list, sp_mod_matrix_list, sp_adj_edge_index,
           W_gcn1, W_comm1, W_gcn2, W_comm2, W_out):
    src = sp_adj_edge_index[0]
    dst = sp_adj_edge_index[1]
    pad = E_PAD - E
    it = jnp.arange(pad, dtype=jnp.int32)
    pad_src = (it * 131) % N
    pad_dst = N + it % (NP - N)
    src_p = jnp.concatenate([src, pad_src]).reshape(NW, CH, 1, CHUNK)
    dst_p = jnp.concatenate([dst, pad_dst]).reshape(NW, CH, 1, CHUNK)
    edges_p = jnp.concatenate([src_p, dst_p], axis=2)
    zeros128 = jnp.zeros((NP, D1), jnp.float32)

    agg = _make_agg()
    x1 = _pc0(nodes_features_list, W_gcn1)
    p1 = agg(x1, edges_p, zeros128)
    comm1 = _pc1(sp_mod_matrix_list, W_comm1)
    x2, comm2 = _pc3(p1, comm1, W_gcn2, W_comm2)
    p2 = agg(x2, edges_p, zeros128)
    x3 = _pc5(p2, comm2, W_out)
    p3 = agg(x3, edges_p, zeros128)
    h = _pc7(p3)
    return (h, comm2)


# speedup vs baseline: 9.2415x; 1.0993x over previous
import functools

import jax
import jax.numpy as jnp
from jax import lax
from jax.experimental import pallas as pl
from jax.experimental.pallas import tpu as pltpu
from jax.experimental.pallas import tpu_sc as plsc

N = 10000
E = 320000
D0 = 128
D1 = 128
D2 = 64

NC = 2
NS = 16
NW = NC * NS
CHUNK = 96
RROW = 3
RIDX = 6
CH = 108
E_PAD = NW * CH * CHUNK
ROWS_PER_TILE = 632
NP = NS * ROWS_PER_TILE


def _leaky(v):
    return jnp.where(v >= 0, v, 0.01 * v)


@functools.lru_cache(None)
def _make_agg():
    mesh = plsc.VectorSubcoreMesh(core_axis_name="c", subcore_axis_name="s")

    @functools.partial(
        pl.kernel,
        mesh=mesh,
        out_type=jax.ShapeDtypeStruct((NC, NP, D1), jnp.float32),
        scratch_types=[
            pltpu.VMEM((RIDX, 2, CHUNK), jnp.int32),
            pltpu.VMEM((RROW, CHUNK, D1), jnp.float32),
            pltpu.VMEM_SHARED((NP, D1), jnp.float32),
        ] + [pltpu.SemaphoreType.DMA] * (2 * RROW + RIDX),
    )
    def agg(x_hbm, src_hbm, dst_hbm, zeros_hbm, out_hbm,
            idx_v, rows_v, acc_sh, *sems):
        gsem = sems[:RROW]
        ssem = sems[RROW:2 * RROW]
        isem = sems[2 * RROW:]
        cid = lax.axis_index("c")
        sid = lax.axis_index("s")
        wid = cid * NS + sid
        row0 = pl.multiple_of(sid * ROWS_PER_TILE, 8)
        pltpu.sync_copy(zeros_hbm.at[pl.ds(row0, ROWS_PER_TILE)],
                        acc_sh.at[pl.ds(row0, ROWS_PER_TILE)])
        plsc.subcore_barrier()

        def _base(j):
            return pl.multiple_of((wid * CH + j) * CHUNK, 8)

        def wait_i(j, r):
            pltpu.make_async_copy(src_hbm.at[pl.ds(_base(j), CHUNK)],
                                  idx_v.at[r, 0], isem[r]).wait()
            pltpu.make_async_copy(dst_hbm.at[pl.ds(_base(j), CHUNK)],
                                  idx_v.at[r, 1], isem[r]).wait()

        def start_i(j, r):
            b0 = _base(j)
            pltpu.async_copy(src_hbm.at[pl.ds(b0, CHUNK)],
                             idx_v.at[r, 0], isem[r])
            pltpu.async_copy(dst_hbm.at[pl.ds(b0, CHUNK)],
                             idx_v.at[r, 1], isem[r])

        def start_g(r, b):
            pltpu.async_copy(x_hbm.at[idx_v.at[r, 0]], rows_v.at[b],
                             gsem[b])

        def wait_g(r, b):
            pltpu.make_async_copy(x_hbm.at[idx_v.at[r, 0]], rows_v.at[b],
                                  gsem[b]).wait()

        def start_s(r, b):
            pltpu.async_copy(rows_v.at[b], acc_sh.at[idx_v.at[r, 1]],
                             ssem[b], add=True)

        def wait_s(r, b):
            pltpu.make_async_copy(rows_v.at[b], acc_sh.at[idx_v.at[r, 1]],
                                  ssem[b]).wait()

        for j in range(RROW):
            start_i(j, j)

        def group(g, carry):
            j0 = g * RIDX
            for u in range(RIDX):
                j = j0 + u
                ri = u
                b = u % RROW
                wait_i(j, ri)

                @pl.when(j >= RROW)
                def _ws(ri=ri, b=b):
                    pltpu.make_async_copy(
                        rows_v.at[b], acc_sh.at[idx_v.at[ri, 1]],
                        ssem[b]).wait()

                jn = j + RROW
                rin = (u + RROW) % RIDX

                @pl.when(jn < CH)
                def _si(jn=jn, rin=rin):
                    start_i(jn, rin)

                start_g(ri, b)

                @pl.when(j >= 2)
                def _ds(u=u, b2=(u - 2) % RROW, ri2=(u - 2) % RIDX):
                    wait_g(ri2, b2)
                    start_s(ri2, b2)
            return carry

        lax.fori_loop(0, CH // RIDX, group, 0)
        for j in (CH - 2, CH - 1):
            ri = j % RIDX
            b = j % RROW
            wait_g(ri, b)
            start_s(ri, b)
        for j in (CH - 3, CH - 2, CH - 1):
            wait_s(j % RIDX, j % RROW)
        plsc.subcore_barrier()
        pltpu.sync_copy(acc_sh.at[pl.ds(row0, ROWS_PER_TILE)],
                        out_hbm.at[cid, pl.ds(row0, ROWS_PER_TILE)])

    return agg


RB1 = 400
G1 = N // RB1
RB = 1000
G = N // RB


def _pc0_body(x_ref, wg_ref, x1_ref):
    x1 = jnp.dot(x_ref[...], wg_ref[...], preferred_element_type=jnp.float32)
    x1_ref[...] = jnp.concatenate(
        [x1, jnp.zeros((NP - N, D1), jnp.float32)], axis=0)


def _pc0(X, Wg):
    return pl.pallas_call(
        _pc0_body,
        grid=(1,),
        in_specs=[
            pl.BlockSpec((N, D0), lambda i: (0, 0)),
            pl.BlockSpec((D0, D1), lambda i: (0, 0)),
        ],
        out_specs=pl.BlockSpec((NP, D1), lambda i: (0, 0)),
        out_shape=jax.ShapeDtypeStruct((NP, D1), jnp.float32),
        compiler_params=pltpu.CompilerParams(
            dimension_semantics=("arbitrary",)),
    )(X, Wg)


def _pc1_body(b_ref, wc_ref, comm_ref):
    comm_ref[...] = _leaky(jnp.dot(b_ref[...], wc_ref[...],
                                   preferred_element_type=jnp.float32))


def _pc1(B, Wc):
    return pl.pallas_call(
        _pc1_body,
        grid=(G1,),
        in_specs=[
            pl.BlockSpec((RB1, N), lambda i: (i, 0)),
            pl.BlockSpec((N, D1), lambda i: (0, 0)),
        ],
        out_specs=pl.BlockSpec((RB1, D1), lambda i: (i, 0)),
        out_shape=jax.ShapeDtypeStruct((N, D1), jnp.float32),
        compiler_params=pltpu.CompilerParams(
            dimension_semantics=("parallel",)),
    )(B, Wc)


def _pc3_body(p_ref, comm1_ref, wg2_ref, wc2_ref, x2_ref, comm2_ref):
    h = _leaky(p_ref[0] + p_ref[1]) + comm1_ref[...]
    x2 = jnp.dot(h, wg2_ref[...], preferred_element_type=jnp.float32)
    x2_ref[...] = jnp.concatenate(
        [x2, jnp.zeros((RB, D1 - D2), jnp.float32)], axis=1)
    comm2_ref[...] = _leaky(jnp.dot(comm1_ref[...], wc2_ref[...],
                                    preferred_element_type=jnp.float32))


def _pc3(p, comm1, Wg2, Wc2):
    return pl.pallas_call(
        _pc3_body,
        grid=(G,),
        in_specs=[
            pl.BlockSpec((NC, RB, D1), lambda i: (0, i, 0)),
            pl.BlockSpec((RB, D1), lambda i: (i, 0)),
            pl.BlockSpec((D1, D2), lambda i: (0, 0)),
            pl.BlockSpec((D1, D2), lambda i: (0, 0)),
        ],
        out_specs=[
            pl.BlockSpec((RB, D1), lambda i: (i, 0)),
            pl.BlockSpec((RB, D2), lambda i: (i, 0)),
        ],
        out_shape=[
            jax.ShapeDtypeStruct((NP, D1), jnp.float32),
            jax.ShapeDtypeStruct((N, D2), jnp.float32),
        ],
        compiler_params=pltpu.CompilerParams(
            dimension_semantics=("parallel",)),
    )(p, comm1, Wg2, Wc2)


def _pc5_body(q_ref, comm2_ref, wout_ref, x3_ref):
    h = _leaky(q_ref[0, :, :D2] + q_ref[1, :, :D2]) + comm2_ref[...]
    x3 = jnp.dot(h, wout_ref[...], preferred_element_type=jnp.float32)
    x3_ref[...] = jnp.concatenate(
        [x3, jnp.zeros((RB, D1 - D2), jnp.float32)], axis=1)


def _pc5(q, comm2, Wout):
    return pl.pallas_call(
        _pc5_body,
        grid=(G,),
        in_specs=[
            pl.BlockSpec((NC, RB, D1), lambda i: (0, i, 0)),
            pl.BlockSpec((RB, D2), lambda i: (i, 0)),
            pl.BlockSpec((D2, D2), lambda i: (0, 0)),
        ],
        out_specs=pl.BlockSpec((RB, D1), lambda i: (i, 0)),
        out_shape=jax.ShapeDtypeStruct((NP, D1), jnp.float32),
        compiler_params=pltpu.CompilerParams(
            dimension_semantics=("parallel",)),
    )(q, comm2, Wout)


def _pc7_body(r_ref, out_ref):
    out_ref[...] = _leaky(r_ref[0, :, :D2] + r_ref[1, :, :D2])


def _pc7(r):
    return pl.pallas_call(
        _pc7_body,
        grid=(G,),
        in_specs=[pl.BlockSpec((NC, RB, D1), lambda i: (0, i, 0))],
        out_specs=pl.BlockSpec((RB, D2), lambda i: (i, 0)),
        out_shape=jax.ShapeDtypeStruct((N, D2), jnp.float32),
        compiler_params=pltpu.CompilerParams(
            dimension_semantics=("parallel",)),
    )(r)


def kernel(nodes_features_list, sp_mod_matrix_list, sp_adj_edge_index,
           W_gcn1, W_comm1, W_gcn2, W_comm2, W_out):
    src = sp_adj_edge_index[0]
    dst = sp_adj_edge_index[1]
    pad = E_PAD - E
    it = jnp.arange(pad, dtype=jnp.int32)
    pad_src = (it * 131) % N
    pad_dst = N + it % (NP - N)
    src_p = jnp.concatenate([src, pad_src])
    dst_p = jnp.concatenate([dst, pad_dst])
    zeros128 = jnp.zeros((NP, D1), jnp.float32)

    agg = _make_agg()
    x1 = _pc0(nodes_features_list, W_gcn1)
    p1 = agg(x1, src_p, dst_p, zeros128)
    comm1 = _pc1(sp_mod_matrix_list, W_comm1)
    x2, comm2 = _pc3(p1, comm1, W_gcn2, W_comm2)
    p2 = agg(x2, src_p, dst_p, zeros128)
    x3 = _pc5(p2, comm2, W_out)
    p3 = agg(x3, src_p, dst_p, zeros128)
    h = _pc7(p3)
    return (h, comm2)


# speedup vs baseline: 9.3604x; 1.0129x over previous
import functools

import jax
import jax.numpy as jnp
from jax import lax
from jax.experimental import pallas as pl
from jax.experimental.pallas import tpu as pltpu
from jax.experimental.pallas import tpu_sc as plsc

N = 10000
E = 320000
D0 = 128
D1 = 128
D2 = 64

NC = 2
NS = 16
NW = NC * NS
CHUNK = 80
RROW = 3
RIDX = 6
CH = 125
CHG = 120
ROWS_PER_TILE = 632
NP = NS * ROWS_PER_TILE


def _leaky(v):
    return jnp.where(v >= 0, v, 0.01 * v)


@functools.lru_cache(None)
def _make_agg():
    mesh = plsc.VectorSubcoreMesh(core_axis_name="c", subcore_axis_name="s")

    @functools.partial(
        pl.kernel,
        mesh=mesh,
        out_type=jax.ShapeDtypeStruct((NC, NP, D1), jnp.float32),
        scratch_types=[
            pltpu.VMEM((RIDX, 2, CHUNK), jnp.int32),
            pltpu.VMEM((RROW, CHUNK, D1), jnp.float32),
            pltpu.VMEM_SHARED((NP, D1), jnp.float32),
        ] + [pltpu.SemaphoreType.DMA] * (2 * RROW + RIDX),
    )
    def agg(x_hbm, src_hbm, dst_hbm, zeros_hbm, out_hbm,
            idx_v, rows_v, acc_sh, *sems):
        gsem = sems[:RROW]
        ssem = sems[RROW:2 * RROW]
        isem = sems[2 * RROW:]
        cid = lax.axis_index("c")
        sid = lax.axis_index("s")
        wid = cid * NS + sid
        row0 = pl.multiple_of(sid * ROWS_PER_TILE, 8)
        pltpu.sync_copy(zeros_hbm.at[pl.ds(row0, ROWS_PER_TILE)],
                        acc_sh.at[pl.ds(row0, ROWS_PER_TILE)])
        plsc.subcore_barrier()

        def _base(j):
            return pl.multiple_of((wid * CH + j) * CHUNK, 8)

        def wait_i(j, r):
            pltpu.make_async_copy(src_hbm.at[pl.ds(_base(j), CHUNK)],
                                  idx_v.at[r, 0], isem[r]).wait()
            pltpu.make_async_copy(dst_hbm.at[pl.ds(_base(j), CHUNK)],
                                  idx_v.at[r, 1], isem[r]).wait()

        def start_i(j, r):
            b0 = _base(j)
            pltpu.async_copy(src_hbm.at[pl.ds(b0, CHUNK)],
                             idx_v.at[r, 0], isem[r])
            pltpu.async_copy(dst_hbm.at[pl.ds(b0, CHUNK)],
                             idx_v.at[r, 1], isem[r])

        def start_g(r, b):
            pltpu.async_copy(x_hbm.at[idx_v.at[r, 0]], rows_v.at[b],
                             gsem[b])

        def wait_g(r, b):
            pltpu.make_async_copy(x_hbm.at[idx_v.at[r, 0]], rows_v.at[b],
                                  gsem[b]).wait()

        def start_s(r, b):
            pltpu.async_copy(rows_v.at[b], acc_sh.at[idx_v.at[r, 1]],
                             ssem[b], add=True)

        def wait_s(r, b):
            pltpu.make_async_copy(rows_v.at[b], acc_sh.at[idx_v.at[r, 1]],
                                  ssem[b]).wait()

        for j in range(RROW):
            start_i(j, j)

        def group(g, carry):
            j0 = g * RIDX
            for u in range(RIDX):
                j = j0 + u
                ri = u
                b = u % RROW
                wait_i(j, ri)

                @pl.when(j >= RROW)
                def _ws(ri=ri, b=b):
                    pltpu.make_async_copy(
                        rows_v.at[b], acc_sh.at[idx_v.at[ri, 1]],
                        ssem[b]).wait()

                jn = j + RROW
                rin = (u + RROW) % RIDX

                @pl.when(jn < CH)
                def _si(jn=jn, rin=rin):
                    start_i(jn, rin)

                start_g(ri, b)

                @pl.when(j >= 2)
                def _ds(u=u, b2=(u - 2) % RROW, ri2=(u - 2) % RIDX):
                    wait_g(ri2, b2)
                    start_s(ri2, b2)
            return carry

        lax.fori_loop(0, CHG // RIDX, group, 0)
        for j in range(CHG, CH):
            ri = j % RIDX
            b = j % RROW
            wait_i(j, ri)
            wait_s(ri, b)
            if j + RROW < CH:
                start_i(j + RROW, (j + RROW) % RIDX)
            start_g(ri, b)
            b2 = (j - 2) % RROW
            ri2 = (j - 2) % RIDX
            wait_g(ri2, b2)
            start_s(ri2, b2)
        for j in (CH - 2, CH - 1):
            ri = j % RIDX
            b = j % RROW
            wait_g(ri, b)
            start_s(ri, b)
        for j in (CH - 3, CH - 2, CH - 1):
            wait_s(j % RIDX, j % RROW)
        plsc.subcore_barrier()
        pltpu.sync_copy(acc_sh.at[pl.ds(row0, ROWS_PER_TILE)],
                        out_hbm.at[cid, pl.ds(row0, ROWS_PER_TILE)])

    return agg


RB1 = 400
G1 = N // RB1
RB = 1000
G = N // RB


def _pc0_body(x_ref, wg_ref, x1_ref):
    x1 = jnp.dot(x_ref[...], wg_ref[...], preferred_element_type=jnp.float32)
    x1_ref[...] = jnp.concatenate(
        [x1, jnp.zeros((NP - N, D1), jnp.float32)], axis=0)


def _pc0(X, Wg):
    return pl.pallas_call(
        _pc0_body,
        grid=(1,),
        in_specs=[
            pl.BlockSpec((N, D0), lambda i: (0, 0)),
            pl.BlockSpec((D0, D1), lambda i: (0, 0)),
        ],
        out_specs=pl.BlockSpec((NP, D1), lambda i: (0, 0)),
        out_shape=jax.ShapeDtypeStruct((NP, D1), jnp.float32),
        compiler_params=pltpu.CompilerParams(
            dimension_semantics=("arbitrary",)),
    )(X, Wg)


def _pc1_body(b_ref, wc_ref, comm_ref):
    comm_ref[...] = _leaky(jnp.dot(b_ref[...], wc_ref[...],
                                   preferred_element_type=jnp.float32))


def _pc1(B, Wc):
    return pl.pallas_call(
        _pc1_body,
        grid=(G1,),
        in_specs=[
            pl.BlockSpec((RB1, N), lambda i: (i, 0)),
            pl.BlockSpec((N, D1), lambda i: (0, 0)),
        ],
        out_specs=pl.BlockSpec((RB1, D1), lambda i: (i, 0)),
        out_shape=jax.ShapeDtypeStruct((N, D1), jnp.float32),
        compiler_params=pltpu.CompilerParams(
            dimension_semantics=("parallel",)),
    )(B, Wc)


def _pc3_body(p_ref, comm1_ref, wg2_ref, wc2_ref, x2_ref, comm2_ref):
    h = _leaky(p_ref[0] + p_ref[1]) + comm1_ref[...]
    x2 = jnp.dot(h, wg2_ref[...], preferred_element_type=jnp.float32)
    x2_ref[...] = jnp.concatenate(
        [x2, jnp.zeros((RB, D1 - D2), jnp.float32)], axis=1)
    comm2_ref[...] = _leaky(jnp.dot(comm1_ref[...], wc2_ref[...],
                                    preferred_element_type=jnp.float32))


def _pc3(p, comm1, Wg2, Wc2):
    return pl.pallas_call(
        _pc3_body,
        grid=(G,),
        in_specs=[
            pl.BlockSpec((NC, RB, D1), lambda i: (0, i, 0)),
            pl.BlockSpec((RB, D1), lambda i: (i, 0)),
            pl.BlockSpec((D1, D2), lambda i: (0, 0)),
            pl.BlockSpec((D1, D2), lambda i: (0, 0)),
        ],
        out_specs=[
            pl.BlockSpec((RB, D1), lambda i: (i, 0)),
            pl.BlockSpec((RB, D2), lambda i: (i, 0)),
        ],
        out_shape=[
            jax.ShapeDtypeStruct((NP, D1), jnp.float32),
            jax.ShapeDtypeStruct((N, D2), jnp.float32),
        ],
        compiler_params=pltpu.CompilerParams(
            dimension_semantics=("parallel",)),
    )(p, comm1, Wg2, Wc2)


def _pc5_body(q_ref, comm2_ref, wout_ref, x3_ref):
    h = _leaky(q_ref[0, :, :D2] + q_ref[1, :, :D2]) + comm2_ref[...]
    x3 = jnp.dot(h, wout_ref[...], preferred_element_type=jnp.float32)
    x3_ref[...] = jnp.concatenate(
        [x3, jnp.zeros((RB, D1 - D2), jnp.float32)], axis=1)


def _pc5(q, comm2, Wout):
    return pl.pallas_call(
        _pc5_body,
        grid=(G,),
        in_specs=[
            pl.BlockSpec((NC, RB, D1), lambda i: (0, i, 0)),
            pl.BlockSpec((RB, D2), lambda i: (i, 0)),
            pl.BlockSpec((D2, D2), lambda i: (0, 0)),
        ],
        out_specs=pl.BlockSpec((RB, D1), lambda i: (i, 0)),
        out_shape=jax.ShapeDtypeStruct((NP, D1), jnp.float32),
        compiler_params=pltpu.CompilerParams(
            dimension_semantics=("parallel",)),
    )(q, comm2, Wout)


def _pc7_body(r_ref, out_ref):
    out_ref[...] = _leaky(r_ref[0, :, :D2] + r_ref[1, :, :D2])


def _pc7(r):
    return pl.pallas_call(
        _pc7_body,
        grid=(G,),
        in_specs=[pl.BlockSpec((NC, RB, D1), lambda i: (0, i, 0))],
        out_specs=pl.BlockSpec((RB, D2), lambda i: (i, 0)),
        out_shape=jax.ShapeDtypeStruct((N, D2), jnp.float32),
        compiler_params=pltpu.CompilerParams(
            dimension_semantics=("parallel",)),
    )(r)


def kernel(nodes_features_list, sp_mod_matrix_list, sp_adj_edge_index,
           W_gcn1, W_comm1, W_gcn2, W_comm2, W_out):
    src_p = sp_adj_edge_index[0]
    dst_p = sp_adj_edge_index[1]
    zeros128 = jnp.zeros((NP, D1), jnp.float32)

    agg = _make_agg()
    x1 = _pc0(nodes_features_list, W_gcn1)
    p1 = agg(x1, src_p, dst_p, zeros128)
    comm1 = _pc1(sp_mod_matrix_list, W_comm1)
    x2, comm2 = _pc3(p1, comm1, W_gcn2, W_comm2)
    p2 = agg(x2, src_p, dst_p, zeros128)
    x3 = _pc5(p2, comm2, W_out)
    p3 = agg(x3, src_p, dst_p, zeros128)
    h = _pc7(p3)
    return (h, comm2)


# speedup vs baseline: 9.3617x; 1.0001x over previous
import functools

import jax
import jax.numpy as jnp
from jax import lax
from jax.experimental import pallas as pl
from jax.experimental.pallas import tpu as pltpu
from jax.experimental.pallas import tpu_sc as plsc

N = 10000
E = 320000
D0 = 128
D1 = 128
D2 = 64

NC = 2
NS = 16
NW = NC * NS
CHUNK = 80
RROW = 3
RIDX = 6
CH = 125
CHG = 120
ROWS_PER_TILE = 632
NP = NS * ROWS_PER_TILE


def _leaky(v):
    return jnp.where(v >= 0, v, 0.01 * v)


@functools.lru_cache(None)
def _make_agg():
    mesh = plsc.VectorSubcoreMesh(core_axis_name="c", subcore_axis_name="s")

    @functools.partial(
        pl.kernel,
        mesh=mesh,
        out_type=jax.ShapeDtypeStruct((NC, NP, D1), jnp.float32),
        scratch_types=[
            pltpu.VMEM((RIDX, 2, CHUNK), jnp.int32),
            pltpu.VMEM((RROW, CHUNK, D1), jnp.float32),
            pltpu.VMEM_SHARED((NP, D1), jnp.float32),
        ] + [pltpu.SemaphoreType.DMA] * (2 * RROW + RIDX),
    )
    def agg(x_hbm, src_hbm, dst_hbm, zeros_hbm, out_hbm,
            idx_v, rows_v, acc_sh, *sems):
        gsem = sems[:RROW]
        ssem = sems[RROW:2 * RROW]
        isem = sems[2 * RROW:]
        cid = lax.axis_index("c")
        sid = lax.axis_index("s")
        wid = cid * NS + sid
        row0 = pl.multiple_of(sid * ROWS_PER_TILE, 8)
        pltpu.sync_copy(zeros_hbm.at[pl.ds(row0, ROWS_PER_TILE)],
                        acc_sh.at[pl.ds(row0, ROWS_PER_TILE)])
        plsc.subcore_barrier()

        def _base(j):
            return pl.multiple_of((wid * CH + j) * CHUNK, 8)

        def wait_i(j, r):
            pltpu.make_async_copy(src_hbm.at[pl.ds(_base(j), CHUNK)],
                                  idx_v.at[r, 0], isem[r]).wait()
            pltpu.make_async_copy(dst_hbm.at[pl.ds(_base(j), CHUNK)],
                                  idx_v.at[r, 1], isem[r]).wait()

        def start_i(j, r):
            b0 = _base(j)
            pltpu.async_copy(src_hbm.at[pl.ds(b0, CHUNK)],
                             idx_v.at[r, 0], isem[r])
            pltpu.async_copy(dst_hbm.at[pl.ds(b0, CHUNK)],
                             idx_v.at[r, 1], isem[r])

        def start_g(r, b):
            pltpu.async_copy(x_hbm.at[idx_v.at[r, 0]], rows_v.at[b],
                             gsem[b])

        def wait_g(r, b):
            pltpu.make_async_copy(x_hbm.at[idx_v.at[r, 0]], rows_v.at[b],
                                  gsem[b]).wait()

        def start_s(r, b):
            pltpu.async_copy(rows_v.at[b], acc_sh.at[idx_v.at[r, 1]],
                             ssem[b], add=True)

        def wait_s(r, b):
            pltpu.make_async_copy(rows_v.at[b], acc_sh.at[idx_v.at[r, 1]],
                                  ssem[b]).wait()

        for j in range(RROW):
            start_i(j, j)

        def group(g, carry):
            j0 = g * RIDX
            for u in range(RIDX):
                j = j0 + u
                ri = u
                b = u % RROW
                wait_i(j, ri)

                @pl.when(j >= RROW)
                def _ws(ri=ri, b=b):
                    pltpu.make_async_copy(
                        rows_v.at[b], acc_sh.at[idx_v.at[ri, 1]],
                        ssem[b]).wait()

                jn = j + RROW
                rin = (u + RROW) % RIDX

                @pl.when(jn < CH)
                def _si(jn=jn, rin=rin):
                    start_i(jn, rin)

                start_g(ri, b)

                @pl.when(j >= 2)
                def _ds(u=u, b2=(u - 2) % RROW, ri2=(u - 2) % RIDX):
                    wait_g(ri2, b2)
                    start_s(ri2, b2)
            return carry

        lax.fori_loop(0, CHG // RIDX, group, 0)
        for j in range(CHG, CH):
            ri = j % RIDX
            b = j % RROW
            wait_i(j, ri)
            wait_s(ri, b)
            if j + RROW < CH:
                start_i(j + RROW, (j + RROW) % RIDX)
            start_g(ri, b)
            b2 = (j - 2) % RROW
            ri2 = (j - 2) % RIDX
            wait_g(ri2, b2)
            start_s(ri2, b2)
        for j in (CH - 2, CH - 1):
            ri = j % RIDX
            b = j % RROW
            wait_g(ri, b)
            start_s(ri, b)
        for j in (CH - 3, CH - 2, CH - 1):
            wait_s(j % RIDX, j % RROW)
        plsc.subcore_barrier()
        pltpu.sync_copy(acc_sh.at[pl.ds(row0, ROWS_PER_TILE)],
                        out_hbm.at[cid, pl.ds(row0, ROWS_PER_TILE)])

    return agg


RB1 = 200
G1 = N // RB1
RB = 1000
G = N // RB


def _pc0_body(x_ref, wg_ref, x1_ref):
    x1 = jnp.dot(x_ref[...], wg_ref[...], preferred_element_type=jnp.float32)
    x1_ref[...] = jnp.concatenate(
        [x1, jnp.zeros((NP - N, D1), jnp.float32)], axis=0)


def _pc0(X, Wg):
    return pl.pallas_call(
        _pc0_body,
        grid=(1,),
        in_specs=[
            pl.BlockSpec((N, D0), lambda i: (0, 0)),
            pl.BlockSpec((D0, D1), lambda i: (0, 0)),
        ],
        out_specs=pl.BlockSpec((NP, D1), lambda i: (0, 0)),
        out_shape=jax.ShapeDtypeStruct((NP, D1), jnp.float32),
        compiler_params=pltpu.CompilerParams(
            dimension_semantics=("arbitrary",)),
    )(X, Wg)


def _pc1_body(b_ref, wc_ref, comm_ref):
    comm_ref[...] = _leaky(jnp.dot(b_ref[...], wc_ref[...],
                                   preferred_element_type=jnp.float32))


def _pc1(B, Wc):
    return pl.pallas_call(
        _pc1_body,
        grid=(G1,),
        in_specs=[
            pl.BlockSpec((RB1, N), lambda i: (i, 0)),
            pl.BlockSpec((N, D1), lambda i: (0, 0)),
        ],
        out_specs=pl.BlockSpec((RB1, D1), lambda i: (i, 0)),
        out_shape=jax.ShapeDtypeStruct((N, D1), jnp.float32),
        compiler_params=pltpu.CompilerParams(
            dimension_semantics=("parallel",)),
    )(B, Wc)


def _pc3_body(p_ref, comm1_ref, wg2_ref, wc2_ref, x2_ref, comm2_ref):
    h = _leaky(p_ref[0] + p_ref[1]) + comm1_ref[...]
    x2 = jnp.dot(h, wg2_ref[...], preferred_element_type=jnp.float32)
    x2_ref[...] = jnp.concatenate(
        [x2, jnp.zeros((RB, D1 - D2), jnp.float32)], axis=1)
    comm2_ref[...] = _leaky(jnp.dot(comm1_ref[...], wc2_ref[...],
                                    preferred_element_type=jnp.float32))


def _pc3(p, comm1, Wg2, Wc2):
    return pl.pallas_call(
        _pc3_body,
        grid=(G,),
        in_specs=[
            pl.BlockSpec((NC, RB, D1), lambda i: (0, i, 0)),
            pl.BlockSpec((RB, D1), lambda i: (i, 0)),
            pl.BlockSpec((D1, D2), lambda i: (0, 0)),
            pl.BlockSpec((D1, D2), lambda i: (0, 0)),
        ],
        out_specs=[
            pl.BlockSpec((RB, D1), lambda i: (i, 0)),
            pl.BlockSpec((RB, D2), lambda i: (i, 0)),
        ],
        out_shape=[
            jax.ShapeDtypeStruct((NP, D1), jnp.float32),
            jax.ShapeDtypeStruct((N, D2), jnp.float32),
        ],
        compiler_params=pltpu.CompilerParams(
            dimension_semantics=("parallel",)),
    )(p, comm1, Wg2, Wc2)


def _pc5_body(q_ref, comm2_ref, wout_ref, x3_ref):
    h = _leaky(q_ref[0, :, :D2] + q_ref[1, :, :D2]) + comm2_ref[...]
    x3 = jnp.dot(h, wout_ref[...], preferred_element_type=jnp.float32)
    x3_ref[...] = jnp.concatenate(
        [x3, jnp.zeros((RB, D1 - D2), jnp.float32)], axis=1)


def _pc5(q, comm2, Wout):
    return pl.pallas_call(
        _pc5_body,
        grid=(G,),
        in_specs=[
            pl.BlockSpec((NC, RB, D1), lambda i: (0, i, 0)),
            pl.BlockSpec((RB, D2), lambda i: (i, 0)),
            pl.BlockSpec((D2, D2), lambda i: (0, 0)),
        ],
        out_specs=pl.BlockSpec((RB, D1), lambda i: (i, 0)),
        out_shape=jax.ShapeDtypeStruct((NP, D1), jnp.float32),
        compiler_params=pltpu.CompilerParams(
            dimension_semantics=("parallel",)),
    )(q, comm2, Wout)


def _pc7_body(r_ref, out_ref):
    out_ref[...] = _leaky(r_ref[0, :, :D2] + r_ref[1, :, :D2])


def _pc7(r):
    return pl.pallas_call(
        _pc7_body,
        grid=(G,),
        in_specs=[pl.BlockSpec((NC, RB, D1), lambda i: (0, i, 0))],
        out_specs=pl.BlockSpec((RB, D2), lambda i: (i, 0)),
        out_shape=jax.ShapeDtypeStruct((N, D2), jnp.float32),
        compiler_params=pltpu.CompilerParams(
            dimension_semantics=("parallel",)),
    )(r)


def kernel(nodes_features_list, sp_mod_matrix_list, sp_adj_edge_index,
           W_gcn1, W_comm1, W_gcn2, W_comm2, W_out):
    src_p = sp_adj_edge_index[0]
    dst_p = sp_adj_edge_index[1]
    zeros128 = jnp.zeros((NP, D1), jnp.float32)

    agg = _make_agg()
    x1 = _pc0(nodes_features_list, W_gcn1)
    p1 = agg(x1, src_p, dst_p, zeros128)
    comm1 = _pc1(sp_mod_matrix_list, W_comm1)
    x2, comm2 = _pc3(p1, comm1, W_gcn2, W_comm2)
    p2 = agg(x2, src_p, dst_p, zeros128)
    x3 = _pc5(p2, comm2, W_out)
    p3 = agg(x3, src_p, dst_p, zeros128)
    h = _pc7(p3)
    return (h, comm2)
